# baseline (speedup 1.0000x reference)
import jax
import jax.numpy as jnp
from jax.experimental import pallas as pl
from jax.experimental.pallas import tpu as pltpu

NROW, NRED, SL, LN = 2048, 256, 32, 128
RB = 32


def _tc_body(idx_ref, vm_ref, red_hbm, o_ref, red_vmem, sem):
    i = pl.program_id(0)

    @pl.when(i == 0)
    def _():
        pltpu.async_copy(red_hbm, red_vmem, sem).wait()

    for r in range(RB):
        j = idx_ref[i * RB + r]
        o_ref[r] = vm_ref[r] + red_vmem[j]


def kernel(V_m, red, vis2red):
    vm3 = V_m.reshape(NROW, SL, LN)
    red3 = red.reshape(NRED, SL, LN)
    rr = jnp.arange(NROW, dtype=jnp.int32)
    p, vis = rr >> 9, rr & 511
    idx = ((p << 6) + vis2red[vis]).astype(jnp.int32)
    grid_spec = pltpu.PrefetchScalarGridSpec(
        num_scalar_prefetch=1,
        grid=(NROW // RB,),
        in_specs=[
            pl.BlockSpec((RB, SL, LN), lambda i, idx: (i, 0, 0)),
            pl.BlockSpec(memory_space=pl.ANY),
        ],
        out_specs=pl.BlockSpec((RB, SL, LN), lambda i, idx: (i, 0, 0)),
        scratch_shapes=[
            pltpu.VMEM((NRED, SL, LN), jnp.float32),
            pltpu.SemaphoreType.DMA,
        ],
    )
    out = pl.pallas_call(
        _tc_body,
        grid_spec=grid_spec,
        out_shape=jax.ShapeDtypeStruct((NROW, SL, LN), jnp.float32),
    )(idx, vm3, red3)
    return out.reshape(V_m.shape)
